# Initial kernel scaffold; baseline (speedup 1.0000x reference)
import jax
import jax.numpy as jnp
from jax.experimental import pallas as pl


def kernel(x, edge_index, batch, W_emb, b_emb, W_c1, b_c1, W_c2, b_c2, W_c3, b_c3, W_k, b_k, W_v, b_v, seed_q, W_o, b_o, W_flat, b_flat):
    raise NotImplementedError("write your pallas kernel here")


# speedup vs baseline: 8.4123x; 8.4123x over previous
import functools

import jax
import jax.numpy as jnp
from jax import lax
from jax.experimental import pallas as pl
from jax.experimental.pallas import tpu as pltpu
from jax.experimental.pallas import tpu_sc as plsc

_N = 10000
_E = 320000
_G = 64
_NP = 10240
_NC = 2
_NS = 16
_NWORK = _NC * _NS
_CH = 128
_K = 80
_KG = 16
_EPAD = _NWORK * _K * _CH
_RPS = _NP // _NS


def _leaky(v):
    return jnp.where(v > 0, v, 0.01 * v)


def _mesh():
    return plsc.VectorSubcoreMesh(
        core_axis_name="c", subcore_axis_name="s",
        num_cores=_NC, num_subcores=_NS)


_SC_PARAMS = pltpu.CompilerParams(use_tc_tiling_on_sc=False)


def _sc_deg(dstp):
    @functools.partial(
        pl.kernel,
        out_type=jax.ShapeDtypeStruct((_NC, _NP), jnp.float32),
        mesh=_mesh(),
        compiler_params=_SC_PARAMS,
        scratch_types=[
            pltpu.VMEM((_K, _CH), jnp.int32),
            pltpu.VMEM((_RPS,), jnp.float32),
            pltpu.VMEM((_CH,), jnp.float32),
            pltpu.VMEM_SHARED((_NP,), jnp.float32),
        ],
    )
    def k(dst_hbm, out_hbm, dst_v, zb_v, ones_v, acc_sh):
        c = lax.axis_index("c")
        s = lax.axis_index("s")
        wid = s * _NC + c

        def zinit(i, carry):
            zb_v[pl.ds(i * 16, 16)] = jnp.zeros((16,), jnp.float32)
            return carry
        lax.fori_loop(0, _RPS // 16, zinit, 0)
        for i in range(_CH // 16):
            ones_v[pl.ds(i * 16, 16)] = jnp.ones((16,), jnp.float32)
        pltpu.sync_copy(zb_v, acc_sh.at[pl.ds(s * _RPS, _RPS)])
        plsc.subcore_barrier()

        pltpu.sync_copy(dst_hbm.at[wid], dst_v)

        def body(j, carry):
            pltpu.sync_copy(ones_v, acc_sh.at[dst_v.at[j]], add=True)
            return carry
        lax.fori_loop(0, _K, body, 0)

        plsc.subcore_barrier()
        pltpu.sync_copy(acc_sh.at[pl.ds(s * _RPS, _RPS)],
                        out_hbm.at[c, pl.ds(s * _RPS, _RPS)])

    return k(dstp)


def _sc_scatter(D):
    @functools.partial(
        pl.kernel,
        out_type=jax.ShapeDtypeStruct((_NC, _NP, D), jnp.float32),
        mesh=_mesh(),
        compiler_params=_SC_PARAMS,
        scratch_types=[
            pltpu.VMEM((_KG, _CH), jnp.int32),
            pltpu.VMEM((_KG, _CH), jnp.int32),
            pltpu.VMEM((2, _CH, D), jnp.float32),
            pltpu.VMEM_SHARED((_NP, D), jnp.float32),
            pltpu.SemaphoreType.DMA,
            pltpu.SemaphoreType.DMA,
        ],
    )
    def k(tp_hbm, src_hbm, dst_hbm, out_hbm,
          src_v, dst_v, rows_v, acc_sh, gsem0, gsem1):
        c = lax.axis_index("c")
        s = lax.axis_index("s")
        wid = s * _NC + c

        def zinit(i, carry):
            for j in range(D // 16):
                rows_v[0, i, pl.ds(j * 16, 16)] = jnp.zeros((16,), jnp.float32)
            return carry
        lax.fori_loop(0, _CH, zinit, 0)
        for t in range(_RPS // _CH):
            pltpu.sync_copy(rows_v.at[0],
                            acc_sh.at[pl.ds(s * _RPS + t * _CH, _CH)])
        plsc.subcore_barrier()

        def group(g, carry):
            pltpu.sync_copy(src_hbm.at[wid, pl.ds(g * _KG, _KG)], src_v)
            pltpu.sync_copy(dst_hbm.at[wid, pl.ds(g * _KG, _KG)], dst_v)
            pltpu.async_copy(tp_hbm.at[src_v.at[0]], rows_v.at[0], gsem0)
            for j in range(_KG):
                if j + 1 < _KG:
                    pltpu.async_copy(
                        tp_hbm.at[src_v.at[j + 1]], rows_v.at[(j + 1) % 2],
                        gsem1 if (j + 1) % 2 else gsem0)
                pltpu.make_async_copy(
                    tp_hbm.at[src_v.at[j]], rows_v.at[j % 2],
                    gsem1 if j % 2 else gsem0).wait()
                pltpu.sync_copy(rows_v.at[j % 2], acc_sh.at[dst_v.at[j]],
                                add=True)
            return carry
        lax.fori_loop(0, _K // _KG, group, 0)

        plsc.subcore_barrier()
        pltpu.sync_copy(acc_sh.at[pl.ds(s * _RPS, _RPS)],
                        out_hbm.at[c, pl.ds(s * _RPS, _RPS)])

    return k


def _rmask(shape):
    return (lax.broadcasted_iota(jnp.int32, shape, 0) < _N).astype(jnp.float32)


def _tc1_body(x_ref, wemb_ref, bemb_ref, d0_ref, d1_ref, wc1_ref,
              tp1_ref, dinv_ref):
    h = jnp.dot(x_ref[...], wemb_ref[...],
                preferred_element_type=jnp.float32) + bemb_ref[...]
    msk = _rmask((_NP, 1))
    hm = h * msk
    mu = jnp.sum(hm, axis=0, keepdims=True) * (1.0 / _N)
    dlt = (h - mu) * msk
    var = jnp.sum(dlt * dlt, axis=0, keepdims=True) * (1.0 / _N)
    sd = jnp.sqrt(var) + 1e-6
    h0 = (h - mu) / sd
    dinv = lax.rsqrt(d0_ref[...] + d1_ref[...] + 1.0)
    t1 = jnp.dot(h0, wc1_ref[...], preferred_element_type=jnp.float32)
    tp1_ref[...] = t1 * dinv * msk
    dinv_ref[...] = dinv


def _tc1(xp, W_emb, b_emb, d0, d1, W_c1):
    return pl.pallas_call(
        _tc1_body,
        out_shape=[jax.ShapeDtypeStruct((_NP, 32), jnp.float32),
                   jax.ShapeDtypeStruct((_NP, 1), jnp.float32)],
    )(xp, W_emb, b_emb, d0, d1, W_c1)


def _tc_mid_body(sa_ref, sb_ref, tp_ref, dinv_ref, b_ref, w_ref, out_ref):
    dinv = dinv_ref[...]
    h = _leaky((sa_ref[...] + sb_ref[...] + tp_ref[...]) * dinv + b_ref[...])
    t = jnp.dot(h, w_ref[...], preferred_element_type=jnp.float32)
    out_ref[...] = t * dinv * _rmask((_NP, 1))


def _tc_mid(sa, sb, tp, dinv, b, w):
    dout = w.shape[1]
    return pl.pallas_call(
        _tc_mid_body,
        out_shape=jax.ShapeDtypeStruct((_NP, dout), jnp.float32),
    )(sa, sb, tp, dinv, b, w)


def _tc4_body(sa_ref, sb_ref, tp_ref, dinv_ref, b_ref, wk_ref, wv_ref,
              bv_ref, tpk_ref, v_ref):
    dinv = dinv_ref[...]
    h3 = _leaky((sa_ref[...] + sb_ref[...] + tp_ref[...]) * dinv + b_ref[...])
    tk = jnp.dot(h3, wk_ref[...], preferred_element_type=jnp.float32)
    tpk_ref[...] = tk * dinv * _rmask((_NP, 1))
    v_ref[...] = jnp.dot(h3, wv_ref[...],
                         preferred_element_type=jnp.float32) + bv_ref[...]


def _tc4(sa, sb, tp, dinv, b, wk, wv, bv):
    return pl.pallas_call(
        _tc4_body,
        out_shape=[jax.ShapeDtypeStruct((_NP, 128), jnp.float32),
                   jax.ShapeDtypeStruct((_NP, 128), jnp.float32)],
    )(sa, sb, tp, dinv, b, wk, wv, bv)


def _tc5_body(sa_ref, sb_ref, tpk_ref, dinv_ref, bk_ref, v_ref, brow_ref,
              q_ref, wo_ref, bo_ref, wf_ref, bf_ref, out_ref):
    kmat = (sa_ref[...] + sb_ref[...] + tpk_ref[...]) * dinv_ref[...] \
        + bk_ref[...]
    rt = (lax.broadcasted_iota(jnp.int32, (128, 8), 0) // 16
          == lax.broadcasted_iota(jnp.int32, (128, 8), 1)).astype(jnp.float32)
    scores = jnp.dot(kmat * q_ref[...], rt,
                     preferred_element_type=jnp.float32) * 0.25
    m = jnp.max(scores, axis=0, keepdims=True)
    ex = jnp.exp(scores - m)
    onehot_t = (brow_ref[...]
                == lax.broadcasted_iota(jnp.int32, (_G, _NP), 0)
                ).astype(jnp.float32)
    den = jnp.dot(onehot_t, ex, preferred_element_type=jnp.float32)
    rexp = (lax.broadcasted_iota(jnp.int32, (8, 128), 0)
            == lax.broadcasted_iota(jnp.int32, (8, 128), 1) // 16
            ).astype(jnp.float32)
    ex_w = jnp.dot(ex, rexp, preferred_element_type=jnp.float32)
    pooled_raw = jnp.dot(onehot_t, ex_w * v_ref[...],
                         preferred_element_type=jnp.float32)
    scale = jnp.dot(den, rexp, preferred_element_type=jnp.float32) + 1e-9
    pooled = pooled_raw / scale
    o = _leaky(jnp.dot(pooled, wo_ref[...],
                       preferred_element_type=jnp.float32) + bo_ref[...])
    out_ref[...] = jnp.dot(o, wf_ref[...],
                           preferred_element_type=jnp.float32) + bf_ref[...]


def _tc5(sa, sb, tpk, dinv, bk, v, brow, q, wo, bo, wf, bf):
    return pl.pallas_call(
        _tc5_body,
        out_shape=jax.ShapeDtypeStruct((_G, 128), jnp.float32),
    )(sa, sb, tpk, dinv, bk, v, brow, q, wo, bo, wf, bf)


def kernel(x, edge_index, batch, W_emb, b_emb, W_c1, b_c1, W_c2, b_c2,
           W_c3, b_c3, W_k, b_k, W_v, b_v, seed_q, W_o, b_o, W_flat, b_flat):
    pad = _EPAD - _E
    padv = jnp.full((pad,), _N, jnp.int32)
    srcp = jnp.concatenate([edge_index[0], padv]).reshape(_NWORK, _K, _CH)
    dstp = jnp.concatenate([edge_index[1], padv]).reshape(_NWORK, _K, _CH)
    xp = jnp.pad(x, ((0, _NP - _N), (0, 0)))
    bpad = jnp.pad(batch, (0, _NP - _N), constant_values=_G)

    degs = _sc_deg(dstp)
    tp1, dinv = _tc1(xp, W_emb, b_emb.reshape(1, -1),
                     degs[0].reshape(_NP, 1), degs[1].reshape(_NP, 1), W_c1)
    s1 = _sc_scatter(32)(tp1, srcp, dstp)
    tp2 = _tc_mid(s1[0], s1[1], tp1, dinv, b_c1.reshape(1, -1), W_c2)
    s2 = _sc_scatter(64)(tp2, srcp, dstp)
    tp3 = _tc_mid(s2[0], s2[1], tp2, dinv, b_c2.reshape(1, -1), W_c3)
    s3 = _sc_scatter(128)(tp3, srcp, dstp)
    tpk, v = _tc4(s3[0], s3[1], tp3, dinv, b_c3.reshape(1, -1),
                  W_k, W_v, b_v.reshape(1, -1))
    sk = _sc_scatter(128)(tpk, srcp, dstp)
    return _tc5(sk[0], sk[1], tpk, dinv, b_k.reshape(1, -1), v,
                bpad.reshape(1, _NP), seed_q.reshape(1, -1),
                W_o, b_o.reshape(1, -1), W_flat, b_flat.reshape(1, -1))


# speedup vs baseline: 20.1102x; 2.3906x over previous
import functools

import jax
import jax.numpy as jnp
from jax import lax
from jax.experimental import pallas as pl
from jax.experimental.pallas import tpu as pltpu
from jax.experimental.pallas import tpu_sc as plsc

_N = 10000
_E = 320000
_G = 64
_NP = 10240
_NC = 2
_NS = 16
_NWORK = _NC * _NS
_CH = 128
_K = 80
_KG = 16
_EPAD = _NWORK * _K * _CH
_RPS = _NP // _NS


def _leaky(v):
    return jnp.where(v > 0, v, 0.01 * v)


def _mesh():
    return plsc.VectorSubcoreMesh(
        core_axis_name="c", subcore_axis_name="s",
        num_cores=_NC, num_subcores=_NS)


_SC_PARAMS = pltpu.CompilerParams(use_tc_tiling_on_sc=False)


def _sc_deg(dstp):
    @functools.partial(
        pl.kernel,
        out_type=jax.ShapeDtypeStruct((_NC, _NP), jnp.float32),
        mesh=_mesh(),
        compiler_params=_SC_PARAMS,
        scratch_types=[
            pltpu.VMEM((_K, _CH), jnp.int32),
            pltpu.VMEM((_RPS,), jnp.float32),
            pltpu.VMEM((_CH,), jnp.float32),
            pltpu.VMEM_SHARED((_NP,), jnp.float32),
        ],
    )
    def k(dst_hbm, out_hbm, dst_v, zb_v, ones_v, acc_sh):
        c = lax.axis_index("c")
        s = lax.axis_index("s")
        wid = s * _NC + c

        def zinit(i, carry):
            zb_v[pl.ds(i * 16, 16)] = jnp.zeros((16,), jnp.float32)
            return carry
        lax.fori_loop(0, _RPS // 16, zinit, 0)
        for i in range(_CH // 16):
            ones_v[pl.ds(i * 16, 16)] = jnp.ones((16,), jnp.float32)
        pltpu.sync_copy(zb_v, acc_sh.at[pl.ds(s * _RPS, _RPS)])
        plsc.subcore_barrier()

        pltpu.sync_copy(dst_hbm.at[wid], dst_v)

        def body(j, carry):
            pltpu.sync_copy(ones_v, acc_sh.at[dst_v.at[j]], add=True)
            return carry
        lax.fori_loop(0, _K, body, 0)

        plsc.subcore_barrier()
        pltpu.sync_copy(acc_sh.at[pl.ds(s * _RPS, _RPS)],
                        out_hbm.at[c, pl.ds(s * _RPS, _RPS)])

    return k(dstp)


def _sc_scatter(D):
    P = 2 if D == 128 else 1
    Dc = D // P

    @functools.partial(
        pl.kernel,
        out_type=jax.ShapeDtypeStruct((_NC, _NP, D), jnp.float32),
        mesh=_mesh(),
        compiler_params=_SC_PARAMS,
        scratch_types=[
            pltpu.VMEM((_K, _CH), jnp.int32),
            pltpu.VMEM((_K, _CH), jnp.int32),
            pltpu.VMEM((2, _CH, Dc), jnp.float32),
            pltpu.VMEM_SHARED((_NP, Dc), jnp.float32),
            pltpu.VMEM_SHARED((_NP, Dc), jnp.float32),
            pltpu.SemaphoreType.DMA,
            pltpu.SemaphoreType.DMA,
            pltpu.SemaphoreType.DMA,
            pltpu.SemaphoreType.DMA,
        ],
    )
    def k(tp_hbm, src_hbm, dst_hbm, out_hbm,
          src_v, dst_v, rows_v, cache_sh, acc_sh, g0, g1, s0, s1):
        gsem = (g0, g1)
        ssem = (s0, s1)
        c = lax.axis_index("c")
        s = lax.axis_index("s")
        wid = s * _NC + c
        row0 = s * _RPS

        pltpu.sync_copy(src_hbm.at[wid], src_v)
        pltpu.sync_copy(dst_hbm.at[wid], dst_v)

        for p in range(P):
            off = p * Dc
            if P == 1:
                pltpu.sync_copy(tp_hbm.at[pl.ds(row0, _RPS)],
                                cache_sh.at[pl.ds(row0, _RPS)])
            else:
                pltpu.sync_copy(tp_hbm.at[pl.ds(row0, _RPS), pl.ds(off, Dc)],
                                cache_sh.at[pl.ds(row0, _RPS)])
            def zinit(i, carry):
                for j in range(Dc // 16):
                    rows_v[0, i, pl.ds(j * 16, 16)] = \
                        jnp.zeros((16,), jnp.float32)
                return carry
            lax.fori_loop(0, _CH, zinit, 0)
            for t in range(_RPS // _CH):
                pltpu.sync_copy(rows_v.at[0],
                                acc_sh.at[pl.ds(row0 + t * _CH, _CH)])
            plsc.subcore_barrier()

            def slot(jj, b, first=False, fire_next=True):
                pltpu.make_async_copy(cache_sh.at[src_v.at[jj]],
                                      rows_v.at[b], gsem[b]).wait()
                pltpu.async_copy(rows_v.at[b], acc_sh.at[dst_v.at[jj]],
                                 ssem[b], add=True)
                nb = 1 - b
                if not first:
                    pltpu.make_async_copy(rows_v.at[nb],
                                          acc_sh.at[dst_v.at[jj - 1]],
                                          ssem[nb]).wait()
                if fire_next:
                    pltpu.async_copy(cache_sh.at[src_v.at[jj + 1]],
                                     rows_v.at[nb], gsem[nb])

            pltpu.async_copy(cache_sh.at[src_v.at[0]], rows_v.at[0], g0)
            slot(0, 0, first=True)

            def pair(g, carry):
                slot(2 * g + 1, 1)
                slot(2 * g + 2, 0)
                return carry
            lax.fori_loop(0, (_K - 2) // 2, pair, 0)
            slot(_K - 1, 1, fire_next=False)
            pltpu.make_async_copy(rows_v.at[1],
                                  acc_sh.at[dst_v.at[_K - 1]], s1).wait()

            plsc.subcore_barrier()
            if P == 1:
                pltpu.sync_copy(acc_sh.at[pl.ds(row0, _RPS)],
                                out_hbm.at[c, pl.ds(row0, _RPS)])
            else:
                pltpu.sync_copy(acc_sh.at[pl.ds(row0, _RPS)],
                                out_hbm.at[c, pl.ds(row0, _RPS),
                                           pl.ds(off, Dc)])
            if p + 1 < P:
                plsc.subcore_barrier()

    return k


def _rmask(shape):
    return (lax.broadcasted_iota(jnp.int32, shape, 0) < _N).astype(jnp.float32)


def _tc1_body(x_ref, wemb_ref, bemb_ref, d0_ref, d1_ref, wc1_ref,
              tp1_ref, dinv_ref):
    h = jnp.dot(x_ref[...], wemb_ref[...],
                preferred_element_type=jnp.float32) + bemb_ref[...]
    msk = _rmask((_NP, 1))
    hm = h * msk
    mu = jnp.sum(hm, axis=0, keepdims=True) * (1.0 / _N)
    dlt = (h - mu) * msk
    var = jnp.sum(dlt * dlt, axis=0, keepdims=True) * (1.0 / _N)
    sd = jnp.sqrt(var) + 1e-6
    h0 = (h - mu) / sd
    dinv = lax.rsqrt(d0_ref[...] + d1_ref[...] + 1.0)
    t1 = jnp.dot(h0, wc1_ref[...], preferred_element_type=jnp.float32)
    tp1_ref[...] = t1 * dinv * msk
    dinv_ref[...] = dinv


def _tc1(xp, W_emb, b_emb, d0, d1, W_c1):
    return pl.pallas_call(
        _tc1_body,
        out_shape=[jax.ShapeDtypeStruct((_NP, 32), jnp.float32),
                   jax.ShapeDtypeStruct((_NP, 1), jnp.float32)],
    )(xp, W_emb, b_emb, d0, d1, W_c1)


def _tc_mid_body(sa_ref, sb_ref, tp_ref, dinv_ref, b_ref, w_ref, out_ref):
    dinv = dinv_ref[...]
    h = _leaky((sa_ref[...] + sb_ref[...] + tp_ref[...]) * dinv + b_ref[...])
    t = jnp.dot(h, w_ref[...], preferred_element_type=jnp.float32)
    out_ref[...] = t * dinv * _rmask((_NP, 1))


def _tc_mid(sa, sb, tp, dinv, b, w):
    dout = w.shape[1]
    return pl.pallas_call(
        _tc_mid_body,
        out_shape=jax.ShapeDtypeStruct((_NP, dout), jnp.float32),
    )(sa, sb, tp, dinv, b, w)


def _tc4_body(sa_ref, sb_ref, tp_ref, dinv_ref, b_ref, wk_ref, wv_ref,
              bv_ref, tpk_ref, v_ref):
    dinv = dinv_ref[...]
    h3 = _leaky((sa_ref[...] + sb_ref[...] + tp_ref[...]) * dinv + b_ref[...])
    tk = jnp.dot(h3, wk_ref[...], preferred_element_type=jnp.float32)
    tpk_ref[...] = tk * dinv * _rmask((_NP, 1))
    v_ref[...] = jnp.dot(h3, wv_ref[...],
                         preferred_element_type=jnp.float32) + bv_ref[...]


def _tc4(sa, sb, tp, dinv, b, wk, wv, bv):
    return pl.pallas_call(
        _tc4_body,
        out_shape=[jax.ShapeDtypeStruct((_NP, 128), jnp.float32),
                   jax.ShapeDtypeStruct((_NP, 128), jnp.float32)],
    )(sa, sb, tp, dinv, b, wk, wv, bv)


def _tc5_body(sa_ref, sb_ref, tpk_ref, dinv_ref, bk_ref, v_ref, brow_ref,
              q_ref, wo_ref, bo_ref, wf_ref, bf_ref, out_ref):
    kmat = (sa_ref[...] + sb_ref[...] + tpk_ref[...]) * dinv_ref[...] \
        + bk_ref[...]
    rt = (lax.broadcasted_iota(jnp.int32, (128, 8), 0) // 16
          == lax.broadcasted_iota(jnp.int32, (128, 8), 1)).astype(jnp.float32)
    scores = jnp.dot(kmat * q_ref[...], rt,
                     preferred_element_type=jnp.float32) * 0.25
    m = jnp.max(scores, axis=0, keepdims=True)
    ex = jnp.exp(scores - m)
    onehot_t = (brow_ref[...]
                == lax.broadcasted_iota(jnp.int32, (_G, _NP), 0)
                ).astype(jnp.float32)
    den = jnp.dot(onehot_t, ex, preferred_element_type=jnp.float32)
    rexp = (lax.broadcasted_iota(jnp.int32, (8, 128), 0)
            == lax.broadcasted_iota(jnp.int32, (8, 128), 1) // 16
            ).astype(jnp.float32)
    ex_w = jnp.dot(ex, rexp, preferred_element_type=jnp.float32)
    pooled_raw = jnp.dot(onehot_t, ex_w * v_ref[...],
                         preferred_element_type=jnp.float32)
    scale = jnp.dot(den, rexp, preferred_element_type=jnp.float32) + 1e-9
    pooled = pooled_raw / scale
    o = _leaky(jnp.dot(pooled, wo_ref[...],
                       preferred_element_type=jnp.float32) + bo_ref[...])
    out_ref[...] = jnp.dot(o, wf_ref[...],
                           preferred_element_type=jnp.float32) + bf_ref[...]


def _tc5(sa, sb, tpk, dinv, bk, v, brow, q, wo, bo, wf, bf):
    return pl.pallas_call(
        _tc5_body,
        out_shape=jax.ShapeDtypeStruct((_G, 128), jnp.float32),
    )(sa, sb, tpk, dinv, bk, v, brow, q, wo, bo, wf, bf)


def kernel(x, edge_index, batch, W_emb, b_emb, W_c1, b_c1, W_c2, b_c2,
           W_c3, b_c3, W_k, b_k, W_v, b_v, seed_q, W_o, b_o, W_flat, b_flat):
    pad = _EPAD - _E
    padv = jnp.full((pad,), _N, jnp.int32)
    srcp = jnp.concatenate([edge_index[0], padv]).reshape(_NWORK, _K, _CH)
    dstp = jnp.concatenate([edge_index[1], padv]).reshape(_NWORK, _K, _CH)
    xp = jnp.pad(x, ((0, _NP - _N), (0, 0)))
    bpad = jnp.pad(batch, (0, _NP - _N), constant_values=_G)

    degs = _sc_deg(dstp)
    tp1, dinv = _tc1(xp, W_emb, b_emb.reshape(1, -1),
                     degs[0].reshape(_NP, 1), degs[1].reshape(_NP, 1), W_c1)
    s1 = _sc_scatter(32)(tp1, srcp, dstp)
    tp2 = _tc_mid(s1[0], s1[1], tp1, dinv, b_c1.reshape(1, -1), W_c2)
    s2 = _sc_scatter(64)(tp2, srcp, dstp)
    tp3 = _tc_mid(s2[0], s2[1], tp2, dinv, b_c2.reshape(1, -1), W_c3)
    s3 = _sc_scatter(128)(tp3, srcp, dstp)
    tpk, v = _tc4(s3[0], s3[1], tp3, dinv, b_c3.reshape(1, -1),
                  W_k, W_v, b_v.reshape(1, -1))
    sk = _sc_scatter(128)(tpk, srcp, dstp)
    return _tc5(sk[0], sk[1], tpk, dinv, b_k.reshape(1, -1), v,
                bpad.reshape(1, _NP), seed_q.reshape(1, -1),
                W_o, b_o.reshape(1, -1), W_flat, b_flat.reshape(1, -1))


# speedup vs baseline: 30.8359x; 1.5334x over previous
import functools

import jax
import jax.numpy as jnp
from jax import lax
from jax.experimental import pallas as pl
from jax.experimental.pallas import tpu as pltpu
from jax.experimental.pallas import tpu_sc as plsc

_N = 10000
_E = 320000
_G = 64
_NP = 10240
_NC = 2
_NS = 16
_NWORK = _NC * _NS
_CH = 128
_K = 80
_KG = 16
_EPAD = _NWORK * _K * _CH
_RPS = _NP // _NS


def _leaky(v):
    return jnp.where(v > 0, v, 0.01 * v)


def _mesh():
    return plsc.VectorSubcoreMesh(
        core_axis_name="c", subcore_axis_name="s",
        num_cores=_NC, num_subcores=_NS)


_SC_PARAMS = pltpu.CompilerParams(use_tc_tiling_on_sc=False)


def _sc_deg(dstp):
    @functools.partial(
        pl.kernel,
        out_type=jax.ShapeDtypeStruct((_NC, _NP), jnp.float32),
        mesh=_mesh(),
        compiler_params=_SC_PARAMS,
        scratch_types=[
            pltpu.VMEM((_K, _CH), jnp.int32),
            pltpu.VMEM((_RPS,), jnp.float32),
            pltpu.VMEM((_CH,), jnp.float32),
            pltpu.VMEM_SHARED((_NP,), jnp.float32),
        ],
    )
    def k(dst_hbm, out_hbm, dst_v, zb_v, ones_v, acc_sh):
        c = lax.axis_index("c")
        s = lax.axis_index("s")
        wid = s * _NC + c

        def zinit(i, carry):
            zb_v[pl.ds(i * 16, 16)] = jnp.zeros((16,), jnp.float32)
            return carry
        lax.fori_loop(0, _RPS // 16, zinit, 0)
        for i in range(_CH // 16):
            ones_v[pl.ds(i * 16, 16)] = jnp.ones((16,), jnp.float32)
        pltpu.sync_copy(zb_v, acc_sh.at[pl.ds(s * _RPS, _RPS)])
        plsc.subcore_barrier()

        pltpu.sync_copy(dst_hbm.at[wid], dst_v)

        def body(j, carry):
            pltpu.sync_copy(ones_v, acc_sh.at[dst_v.at[j]], add=True)
            return carry
        lax.fori_loop(0, _K, body, 0)

        plsc.subcore_barrier()
        pltpu.sync_copy(acc_sh.at[pl.ds(s * _RPS, _RPS)],
                        out_hbm.at[c, pl.ds(s * _RPS, _RPS)])

    return k(dstp)


def _sc_scatter(D):
    P = 2 if D == 128 else 1
    Dc = D // P

    @functools.partial(
        pl.kernel,
        out_type=jax.ShapeDtypeStruct((_NC, _NP, D), jnp.float32),
        mesh=_mesh(),
        compiler_params=_SC_PARAMS,
        scratch_types=[
            pltpu.VMEM((_K, _CH), jnp.int32),
            pltpu.VMEM((_K, _CH), jnp.int32),
            pltpu.VMEM((2, _CH, Dc), jnp.float32),
            pltpu.VMEM_SHARED((_NP, Dc), jnp.float32),
            pltpu.VMEM_SHARED((_NP, Dc), jnp.float32),
            pltpu.SemaphoreType.DMA,
            pltpu.SemaphoreType.DMA,
            pltpu.SemaphoreType.DMA,
            pltpu.SemaphoreType.DMA,
        ],
    )
    def k(tp_hbm, src_hbm, dst_hbm, out_hbm,
          src_v, dst_v, rows_v, cache_sh, acc_sh, g0, g1, s0, s1):
        gsem = (g0, g1)
        ssem = (s0, s1)
        c = lax.axis_index("c")
        s = lax.axis_index("s")
        wid = s * _NC + c
        row0 = s * _RPS

        pltpu.sync_copy(src_hbm.at[wid], src_v)
        pltpu.sync_copy(dst_hbm.at[wid], dst_v)

        for p in range(P):
            off = p * Dc
            if P == 1:
                pltpu.sync_copy(tp_hbm.at[pl.ds(row0, _RPS)],
                                cache_sh.at[pl.ds(row0, _RPS)])
            else:
                pltpu.sync_copy(tp_hbm.at[pl.ds(row0, _RPS), pl.ds(off, Dc)],
                                cache_sh.at[pl.ds(row0, _RPS)])
            def zinit(i, carry):
                for j in range(Dc // 16):
                    rows_v[0, i, pl.ds(j * 16, 16)] = \
                        jnp.zeros((16,), jnp.float32)
                return carry
            lax.fori_loop(0, _CH, zinit, 0)
            for t in range(_RPS // _CH):
                pltpu.sync_copy(rows_v.at[0],
                                acc_sh.at[pl.ds(row0 + t * _CH, _CH)])
            plsc.subcore_barrier()

            def slot(jj, b, first=False, fire_next=True):
                pltpu.make_async_copy(cache_sh.at[src_v.at[jj]],
                                      rows_v.at[b], gsem[b]).wait()
                pltpu.async_copy(rows_v.at[b], acc_sh.at[dst_v.at[jj]],
                                 ssem[b], add=True)
                nb = 1 - b
                if not first:
                    pltpu.make_async_copy(rows_v.at[nb],
                                          acc_sh.at[dst_v.at[jj - 1]],
                                          ssem[nb]).wait()
                if fire_next:
                    pltpu.async_copy(cache_sh.at[src_v.at[jj + 1]],
                                     rows_v.at[nb], gsem[nb])

            pltpu.async_copy(cache_sh.at[src_v.at[0]], rows_v.at[0], g0)
            slot(0, 0, first=True)

            def pair(g, carry):
                slot(2 * g + 1, 1)
                slot(2 * g + 2, 0)
                return carry
            lax.fori_loop(0, (_K - 2) // 2, pair, 0)
            slot(_K - 1, 1, fire_next=False)
            pltpu.make_async_copy(rows_v.at[1],
                                  acc_sh.at[dst_v.at[_K - 1]], s1).wait()

            plsc.subcore_barrier()
            if P == 1:
                pltpu.sync_copy(acc_sh.at[pl.ds(row0, _RPS)],
                                out_hbm.at[c, pl.ds(row0, _RPS)])
            else:
                pltpu.sync_copy(acc_sh.at[pl.ds(row0, _RPS)],
                                out_hbm.at[c, pl.ds(row0, _RPS),
                                           pl.ds(off, Dc)])
            if p + 1 < P:
                plsc.subcore_barrier()

    return k


def _rmask(shape):
    return (lax.broadcasted_iota(jnp.int32, shape, 0) < _N).astype(jnp.float32)


def _tc1_body(x_ref, wemb_ref, bemb_ref, d0_ref, d1_ref, wc1_ref,
              tp1_ref, dinv_ref):
    h = jnp.dot(x_ref[...], wemb_ref[...],
                preferred_element_type=jnp.float32) + bemb_ref[...]
    msk = _rmask((_NP, 1))
    hm = h * msk
    mu = jnp.sum(hm, axis=0, keepdims=True) * (1.0 / _N)
    dlt = (h - mu) * msk
    var = jnp.sum(dlt * dlt, axis=0, keepdims=True) * (1.0 / _N)
    sd = jnp.sqrt(var) + 1e-6
    h0 = (h - mu) / sd
    dinv = lax.rsqrt(d0_ref[...] + d1_ref[...] + 1.0)
    t1 = jnp.dot(h0, wc1_ref[...], preferred_element_type=jnp.float32)
    tp1_ref[...] = t1 * dinv * msk
    dinv_ref[...] = dinv


def _tc1(xp, W_emb, b_emb, d0, d1, W_c1):
    return pl.pallas_call(
        _tc1_body,
        out_shape=[jax.ShapeDtypeStruct((_NP, 32), jnp.float32),
                   jax.ShapeDtypeStruct((_NP, 1), jnp.float32)],
    )(xp, W_emb, b_emb, d0, d1, W_c1)


def _tc2_body(sa_ref, sb_ref, tp_ref, dinv_ref, b_ref, out_ref):
    dinv = dinv_ref[...]
    h = _leaky((sa_ref[...] + sb_ref[...] + tp_ref[...]) * dinv + b_ref[...])
    out_ref[...] = h * dinv * _rmask((_NP, 1))


def _tc2(sa, sb, tp, dinv, b):
    return pl.pallas_call(
        _tc2_body,
        out_shape=jax.ShapeDtypeStruct((_NP, 32), jnp.float32),
    )(sa, sb, tp, dinv, b)


def _tc3_body(za_ref, zb_ref, y_ref, dinv_ref, b_ref, w_ref, out_ref):
    dinv = dinv_ref[...]
    agg = za_ref[...] + zb_ref[...] + y_ref[...]
    h = _leaky(jnp.dot(agg, w_ref[...],
                       preferred_element_type=jnp.float32) * dinv + b_ref[...])
    out_ref[...] = h * dinv * _rmask((_NP, 1))


def _tc3(za, zb, y, dinv, b, w):
    dout = w.shape[1]
    return pl.pallas_call(
        _tc3_body,
        out_shape=jax.ShapeDtypeStruct((_NP, dout), jnp.float32),
    )(za, zb, y, dinv, b, w)


def _tc4_body(za_ref, zb_ref, y_ref, dinv_ref, b_ref, w_ref, wk_ref,
              qcol_ref, wv_ref, bv_ref, w16_ref, v_ref):
    dinv = dinv_ref[...]
    agg = za_ref[...] + zb_ref[...] + y_ref[...]
    h3 = _leaky(jnp.dot(agg, w_ref[...],
                        preferred_element_type=jnp.float32) * dinv
                + b_ref[...])
    rt = (lax.broadcasted_iota(jnp.int32, (128, 16), 0) // 16
          == lax.broadcasted_iota(jnp.int32, (128, 16), 1)).astype(jnp.float32)
    wq = jnp.dot(wk_ref[...], qcol_ref[...] * rt,
                 preferred_element_type=jnp.float32)
    msk = _rmask((_NP, 1))
    w16_ref[...] = jnp.dot(h3 * dinv, wq,
                           preferred_element_type=jnp.float32) * msk
    v_ref[...] = jnp.dot(h3, wv_ref[...],
                         preferred_element_type=jnp.float32) + bv_ref[...]


def _tc4(za, zb, y, dinv, b, w, wk, qcol, wv, bv):
    return pl.pallas_call(
        _tc4_body,
        out_shape=[jax.ShapeDtypeStruct((_NP, 16), jnp.float32),
                   jax.ShapeDtypeStruct((_NP, 128), jnp.float32)],
    )(za, zb, y, dinv, b, w, wk, qcol, wv, bv)


def _tc5_body(za_ref, zb_ref, w16_ref, dinv_ref, bk_ref, v_ref, brow_ref,
              qcol_ref, wo_ref, bo_ref, wf_ref, bf_ref, out_ref):
    rt = (lax.broadcasted_iota(jnp.int32, (128, 8), 0) // 16
          == lax.broadcasted_iota(jnp.int32, (128, 8), 1)).astype(jnp.float32)
    bq = jnp.dot(bk_ref[...], qcol_ref[...] * rt,
                 preferred_element_type=jnp.float32)
    zsum = (za_ref[...] + zb_ref[...] + w16_ref[...])[:, :8]
    scores = zsum * dinv_ref[...] * 0.25 + bq
    m = jnp.max(scores, axis=0, keepdims=True)
    ex = jnp.exp(scores - m)
    onehot_t = (brow_ref[...]
                == lax.broadcasted_iota(jnp.int32, (_G, _NP), 0)
                ).astype(jnp.float32)
    den = jnp.dot(onehot_t, ex, preferred_element_type=jnp.float32)
    rexp = (lax.broadcasted_iota(jnp.int32, (8, 128), 0)
            == lax.broadcasted_iota(jnp.int32, (8, 128), 1) // 16
            ).astype(jnp.float32)
    ex_w = jnp.dot(ex, rexp, preferred_element_type=jnp.float32)
    pooled_raw = jnp.dot(onehot_t, ex_w * v_ref[...],
                         preferred_element_type=jnp.float32)
    scale = jnp.dot(den, rexp, preferred_element_type=jnp.float32) + 1e-9
    pooled = pooled_raw / scale
    o = _leaky(jnp.dot(pooled, wo_ref[...],
                       preferred_element_type=jnp.float32) + bo_ref[...])
    out_ref[...] = jnp.dot(o, wf_ref[...],
                           preferred_element_type=jnp.float32) + bf_ref[...]


def _tc5(za, zb, w16, dinv, bk, v, brow, qcol, wo, bo, wf, bf):
    return pl.pallas_call(
        _tc5_body,
        out_shape=jax.ShapeDtypeStruct((_G, 128), jnp.float32),
    )(za, zb, w16, dinv, bk, v, brow, qcol, wo, bo, wf, bf)


def kernel(x, edge_index, batch, W_emb, b_emb, W_c1, b_c1, W_c2, b_c2,
           W_c3, b_c3, W_k, b_k, W_v, b_v, seed_q, W_o, b_o, W_flat, b_flat):
    pad = _EPAD - _E
    padv = jnp.full((pad,), _N, jnp.int32)
    srcp = jnp.concatenate([edge_index[0], padv]).reshape(_NWORK, _K, _CH)
    dstp = jnp.concatenate([edge_index[1], padv]).reshape(_NWORK, _K, _CH)
    xp = jnp.pad(x, ((0, _NP - _N), (0, 0)))
    bpad = jnp.pad(batch, (0, _NP - _N), constant_values=_G)

    qcol = seed_q.reshape(-1, 1)
    degs = _sc_deg(dstp)
    tp1, dinv = _tc1(xp, W_emb, b_emb.reshape(1, -1),
                     degs[0].reshape(_NP, 1), degs[1].reshape(_NP, 1), W_c1)
    s1 = _sc_scatter(32)(tp1, srcp, dstp)
    y2 = _tc2(s1[0], s1[1], tp1, dinv, b_c1.reshape(1, -1))
    z2 = _sc_scatter(32)(y2, srcp, dstp)
    y3 = _tc3(z2[0], z2[1], y2, dinv, b_c2.reshape(1, -1), W_c2)
    z3 = _sc_scatter(64)(y3, srcp, dstp)
    w16, v = _tc4(z3[0], z3[1], y3, dinv, b_c3.reshape(1, -1), W_c3,
                  W_k, qcol, W_v, b_v.reshape(1, -1))
    zk = _sc_scatter(16)(w16, srcp, dstp)
    return _tc5(zk[0], zk[1], w16, dinv, b_k.reshape(1, -1), v,
                bpad.reshape(1, _NP), qcol,
                W_o, b_o.reshape(1, -1), W_flat, b_flat.reshape(1, -1))
